# baseline (speedup 1.0000x reference)
import functools

import jax
import jax.numpy as jnp
from jax import lax
from jax.experimental import pallas as pl
from jax.experimental.pallas import tpu as pltpu
from jax.experimental.pallas import tpu_sc as plsc

N_NODES = 256
N_OUT = 64
BATCH = 16384
LANES = 16
NUM_WORKERS = 16
ROWS_PER_WORKER = BATCH // NUM_WORKERS
CHUNK = 128
N_GROUPS = N_OUT // LANES

_mesh = plsc.VectorSubcoreMesh(
    core_axis_name="c", subcore_axis_name="s", num_cores=1
)


@functools.partial(
    pl.kernel,
    mesh=_mesh,
    out_type=jax.ShapeDtypeStruct((BATCH, N_OUT), jnp.float32),
    compiler_params=pltpu.CompilerParams(
        needs_layout_passes=False,
    ),
    scratch_types=[
        pltpu.VMEM((N_OUT,), jnp.int32),
        pltpu.VMEM((N_OUT,), jnp.float32),
        pltpu.VMEM((CHUNK, N_NODES), jnp.float32),
        pltpu.VMEM((CHUNK, N_OUT), jnp.float32),
    ],
)
def _gather_scale(y_hbm, w_hbm, idx_hbm, out_hbm, idx_v, w_v, in_v, out_v):
    wid = lax.axis_index("s")
    row0 = wid * ROWS_PER_WORKER

    pltpu.sync_copy(idx_hbm, idx_v)
    pltpu.sync_copy(w_hbm, w_v)

    idx_vecs = [idx_v[pl.ds(g * LANES, LANES)] for g in range(N_GROUPS)]
    w_vecs = [w_v[pl.ds(g * LANES, LANES)] for g in range(N_GROUPS)]

    for c in range(ROWS_PER_WORKER // CHUNK):
        base_row = row0 + c * CHUNK
        pltpu.sync_copy(y_hbm.at[pl.ds(base_row, CHUNK)], in_v)

        def body(r, _):
            row_idx = lax.broadcast(r, (LANES,))
            for g in range(N_GROUPS):
                vals = plsc.load_gather(in_v, [row_idx, idx_vecs[g]])
                out_v[r, pl.ds(g * LANES, LANES)] = vals * w_vecs[g]
            return _

        lax.fori_loop(0, CHUNK, body, None)

        pltpu.sync_copy(out_v, out_hbm.at[pl.ds(base_row, CHUNK)])


def kernel(Y_full, weights, output_node_order):
    return _gather_scale(Y_full, weights, output_node_order)


# speedup vs baseline: 1.6169x; 1.6169x over previous
import functools

import jax
import jax.numpy as jnp
from jax import lax
from jax.experimental import pallas as pl
from jax.experimental.pallas import tpu as pltpu
from jax.experimental.pallas import tpu_sc as plsc

N_NODES = 256
N_OUT = 64
BATCH = 16384
LANES = 16
NUM_WORKERS = 32
ROWS_PER_WORKER = BATCH // NUM_WORKERS
CHUNK = 128
N_CHUNKS = ROWS_PER_WORKER // CHUNK
N_GROUPS = N_OUT // LANES
NBUF = 2

_mesh = plsc.VectorSubcoreMesh(core_axis_name="c", subcore_axis_name="s")


@functools.partial(
    pl.kernel,
    mesh=_mesh,
    out_type=jax.ShapeDtypeStruct((BATCH, N_OUT), jnp.float32),
    compiler_params=pltpu.CompilerParams(
        needs_layout_passes=False,
    ),
    scratch_types=[
        pltpu.VMEM((N_OUT,), jnp.int32),
        pltpu.VMEM((N_OUT,), jnp.float32),
        pltpu.VMEM((NBUF, CHUNK, N_NODES), jnp.float32),
        pltpu.VMEM((NBUF, CHUNK, N_OUT), jnp.float32),
        pltpu.SemaphoreType.DMA,
        pltpu.SemaphoreType.DMA,
    ],
)
def _gather_scale(
    y_hbm, w_hbm, idx_hbm, out_hbm, idx_v, w_v, in_v, out_v, in_sem, out_sem
):
    wid = lax.axis_index("s") * 2 + lax.axis_index("c")
    row0 = wid * ROWS_PER_WORKER

    pltpu.sync_copy(idx_hbm, idx_v)
    pltpu.sync_copy(w_hbm, w_v)

    idx_vecs = [idx_v[pl.ds(g * LANES, LANES)] for g in range(N_GROUPS)]
    w_vecs = [w_v[pl.ds(g * LANES, LANES)] for g in range(N_GROUPS)]

    def start_in(c, buf):
        pltpu.async_copy(
            y_hbm.at[pl.ds(row0 + c * CHUNK, CHUNK)], in_v.at[buf], in_sem
        )

    start_in(0, 0)
    for c in range(N_CHUNKS):
        buf = c % NBUF
        if c + 1 < N_CHUNKS:
            start_in(c + 1, (c + 1) % NBUF)
        pltpu.make_async_copy(
            y_hbm.at[pl.ds(row0, CHUNK)], in_v.at[buf], in_sem
        ).wait()
        if c >= NBUF:
            pltpu.make_async_copy(
                out_v.at[buf], out_hbm.at[pl.ds(row0, CHUNK)], out_sem
            ).wait()

        in_c = in_v.at[buf]
        out_c = out_v.at[buf]

        def body(r, _):
            row_idx = lax.broadcast(r, (LANES,))
            for g in range(N_GROUPS):
                vals = plsc.load_gather(in_c, [row_idx, idx_vecs[g]])
                out_c[r, pl.ds(g * LANES, LANES)] = vals * w_vecs[g]
            return _

        lax.fori_loop(0, CHUNK, body, None)

        pltpu.async_copy(
            out_c, out_hbm.at[pl.ds(row0 + c * CHUNK, CHUNK)], out_sem
        )

    for _ in range(min(NBUF, N_CHUNKS)):
        pltpu.make_async_copy(
            out_v.at[0], out_hbm.at[pl.ds(row0, CHUNK)], out_sem
        ).wait()


def kernel(Y_full, weights, output_node_order):
    return _gather_scale(Y_full, weights, output_node_order)


# speedup vs baseline: 1.7857x; 1.1044x over previous
import functools

import jax
import jax.numpy as jnp
from jax import lax
from jax.experimental import pallas as pl
from jax.experimental.pallas import tpu as pltpu
from jax.experimental.pallas import tpu_sc as plsc

N_NODES = 256
N_OUT = 64
BATCH = 16384
LANES = 16
NUM_WORKERS = 32
ROWS_PER_WORKER = BATCH // NUM_WORKERS
CHUNK = 128
N_CHUNKS = ROWS_PER_WORKER // CHUNK
N_GROUPS = N_OUT // LANES
NBUF = 2

_mesh = plsc.VectorSubcoreMesh(core_axis_name="c", subcore_axis_name="s")


@functools.partial(
    pl.kernel,
    mesh=_mesh,
    out_type=jax.ShapeDtypeStruct((BATCH, N_OUT), jnp.float32),
    compiler_params=pltpu.CompilerParams(
        needs_layout_passes=False,
    ),
    scratch_types=[
        pltpu.VMEM((N_OUT,), jnp.int32),
        pltpu.VMEM((N_OUT,), jnp.float32),
        pltpu.VMEM((NBUF, CHUNK, N_NODES), jnp.float32),
        pltpu.VMEM((NBUF, CHUNK, N_OUT), jnp.float32),
        pltpu.SemaphoreType.DMA,
        pltpu.SemaphoreType.DMA,
    ],
)
def _gather_scale(
    y_hbm, w_hbm, idx_hbm, out_hbm, idx_v, w_v, in_v, out_v, in_sem, out_sem
):
    wid = lax.axis_index("s") * 2 + lax.axis_index("c")
    row0 = wid * ROWS_PER_WORKER

    pltpu.sync_copy(idx_hbm, idx_v)
    pltpu.sync_copy(w_hbm, w_v)

    idx_vecs = [idx_v[pl.ds(g * LANES, LANES)] for g in range(N_GROUPS)]
    w_vecs = [w_v[pl.ds(g * LANES, LANES)] for g in range(N_GROUPS)]

    def start_in(c, buf):
        pltpu.async_copy(
            y_hbm.at[pl.ds(row0 + c * CHUNK, CHUNK)], in_v.at[buf], in_sem
        )

    start_in(0, 0)
    for c in range(N_CHUNKS):
        buf = c % NBUF
        if c + 1 < N_CHUNKS:
            start_in(c + 1, (c + 1) % NBUF)
        pltpu.make_async_copy(
            y_hbm.at[pl.ds(row0, CHUNK)], in_v.at[buf], in_sem
        ).wait()
        if c >= NBUF:
            pltpu.make_async_copy(
                out_v.at[buf], out_hbm.at[pl.ds(row0, CHUNK)], out_sem
            ).wait()

        in_c = in_v.at[buf]
        out_c = out_v.at[buf]

        @plsc.parallel_loop(0, CHUNK, unroll=4)
        def body(r):
            row_idx = lax.broadcast(r, (LANES,))
            for g in range(N_GROUPS):
                vals = plsc.load_gather(in_c, [row_idx, idx_vecs[g]])
                out_c[r, pl.ds(g * LANES, LANES)] = vals * w_vecs[g]

        pltpu.async_copy(
            out_c, out_hbm.at[pl.ds(row0 + c * CHUNK, CHUNK)], out_sem
        )

    for _ in range(min(NBUF, N_CHUNKS)):
        pltpu.make_async_copy(
            out_v.at[0], out_hbm.at[pl.ds(row0, CHUNK)], out_sem
        ).wait()


def kernel(Y_full, weights, output_node_order):
    return _gather_scale(Y_full, weights, output_node_order)
